# baseline (speedup 1.0000x reference)
import functools

import jax
import jax.numpy as jnp
from jax import lax
from jax.experimental import pallas as pl
from jax.experimental.pallas import tpu as pltpu
from jax.experimental.pallas import tpu_sc as plsc

_CH = 1024
_NBUF = 3


@functools.cache
def _make_gather(V, D, N):
    info = plsc.get_sparse_core_info()
    NC, NS = info.num_cores, info.num_subcores
    NW = NC * NS
    assert N % (NW * _CH) == 0
    b_per_w = N // NW
    n_ch = b_per_w // _CH
    mesh = plsc.VectorSubcoreMesh(core_axis_name="c", subcore_axis_name="s")

    @functools.partial(
        pl.kernel,
        mesh=mesh,
        out_type=jax.ShapeDtypeStruct((N, D), jnp.float32),
        scratch_types=[
            pltpu.VMEM((n_ch, _CH), jnp.int32),
            pltpu.VMEM((_NBUF, _CH, D), jnp.float32),
        ]
        + [pltpu.SemaphoreType.DMA] * (2 * _NBUF),
        compiler_params=pltpu.CompilerParams(use_tc_tiling_on_sc=False),
    )
    def gather_kernel(idx_hbm, table_hbm, out_hbm, idx_all, rows, *sems):
        sem_g, sem_w = sems[:_NBUF], sems[_NBUF:]
        wid = lax.axis_index("s") * NC + lax.axis_index("c")
        base = wid * b_per_w
        pltpu.sync_copy(idx_hbm.at[wid], idx_all)

        gathers = {}
        for b in range(min(_NBUF, n_ch)):
            gathers[b] = pltpu.async_copy(
                table_hbm.at[idx_all.at[b]], rows.at[b], sem_g[b])
        for i in range(n_ch):
            b = i % _NBUF
            gathers[i].wait()
            wb = pltpu.async_copy(
                rows.at[b], out_hbm.at[pl.ds(base + i * _CH, _CH)], sem_w[b])
            nxt = i + _NBUF
            wb.wait()
            if nxt < n_ch:
                gathers[nxt] = pltpu.async_copy(
                    table_hbm.at[idx_all.at[nxt]], rows.at[b], sem_g[b])

    return gather_kernel


def _unflatten_body(in_ref, out_ref):
    out_ref[...] = in_ref[...].reshape(out_ref.shape)


@functools.cache
def _make_unflatten(B, F, D):
    T = 512
    assert B % T == 0
    return pl.pallas_call(
        _unflatten_body,
        grid=(B // T,),
        in_specs=[pl.BlockSpec((T * F, D), lambda i: (i, 0))],
        out_specs=pl.BlockSpec((T, F, D), lambda i: (i, 0, 0)),
        out_shape=jax.ShapeDtypeStruct((B, F, D), jnp.float32),
    )


def kernel(token_ids, weight):
    B, F = token_ids.shape
    V, D = weight.shape
    N = B * F
    info = plsc.get_sparse_core_info()
    NW = info.num_cores * info.num_subcores
    idx = token_ids.reshape(NW, N // (NW * _CH), _CH)
    flat = _make_gather(V, D, N)(idx, weight)
    return _make_unflatten(B, F, D)(flat)


# speedup vs baseline: 1.3263x; 1.3263x over previous
import functools

import jax
import jax.numpy as jnp
from jax import lax
from jax.experimental import pallas as pl
from jax.experimental.pallas import tpu as pltpu
from jax.experimental.pallas import tpu_sc as plsc

_CH = 1024
_NBUF = 3


@functools.cache
def _make_gather(V, D, N):
    info = plsc.get_sparse_core_info()
    NC, NS = info.num_cores, info.num_subcores
    NW = NC * NS
    assert N % (NW * _CH) == 0
    b_per_w = N // NW
    n_ch = b_per_w // _CH
    mesh = plsc.VectorSubcoreMesh(core_axis_name="c", subcore_axis_name="s")

    @functools.partial(
        pl.kernel,
        mesh=mesh,
        out_type=jax.ShapeDtypeStruct((N, D), jnp.float32),
        scratch_types=[
            pltpu.VMEM((n_ch, _CH), jnp.int32),
            pltpu.VMEM((_NBUF, _CH, D), jnp.float32),
        ]
        + [pltpu.SemaphoreType.DMA] * (2 * _NBUF),
        compiler_params=pltpu.CompilerParams(use_tc_tiling_on_sc=False),
    )
    def gather_kernel(idx_hbm, table_hbm, out_hbm, idx_all, rows, *sems):
        sem_g, sem_w = sems[:_NBUF], sems[_NBUF:]
        wid = lax.axis_index("s") * NC + lax.axis_index("c")
        base = wid * b_per_w
        pltpu.sync_copy(idx_hbm.at[wid], idx_all)

        gathers = {}
        for b in range(min(_NBUF, n_ch)):
            gathers[b] = pltpu.async_copy(
                table_hbm.at[idx_all.at[b]], rows.at[b], sem_g[b])
        for i in range(n_ch):
            b = i % _NBUF
            gathers[i].wait()
            wb = pltpu.async_copy(
                rows.at[b], out_hbm.at[pl.ds(base + i * _CH, _CH)], sem_w[b])
            nxt = i + _NBUF
            wb.wait()
            if nxt < n_ch:
                gathers[nxt] = pltpu.async_copy(
                    table_hbm.at[idx_all.at[nxt]], rows.at[b], sem_g[b])

    return gather_kernel


def _transpose_body(in_ref, out_ref):
    out_ref[...] = jnp.transpose(in_ref[...], (0, 2, 1))


@functools.cache
def _make_plane_transpose(B, F, D):
    return pl.pallas_call(
        _transpose_body,
        grid=(F,),
        in_specs=[pl.BlockSpec((1, B, D), lambda f: (f, 0, 0))],
        out_specs=pl.BlockSpec((1, D, B), lambda f: (f, 0, 0)),
        out_shape=jax.ShapeDtypeStruct((F, D, B), jnp.float32),
    )


def kernel(token_ids, weight):
    B, F = token_ids.shape
    V, D = weight.shape
    N = B * F
    info = plsc.get_sparse_core_info()
    NW = info.num_cores * info.num_subcores
    idx = token_ids.T.reshape(NW, N // (NW * _CH), _CH)
    flat = _make_gather(V, D, N)(idx, weight)
    planes = _make_plane_transpose(B, F, D)(flat.reshape(F, B, D))
    return planes.transpose(2, 0, 1)


# speedup vs baseline: 1.7799x; 1.3420x over previous
import functools

import jax
import jax.numpy as jnp
from jax import lax
from jax.experimental import pallas as pl
from jax.experimental.pallas import tpu as pltpu
from jax.experimental.pallas import tpu_sc as plsc

_CH = 1024
_NBUF = 3


@functools.cache
def _make_gather(V, D, N):
    info = plsc.get_sparse_core_info()
    NC, NS = info.num_cores, info.num_subcores
    NW = NC * NS
    assert N % (NW * _CH) == 0
    b_per_w = N // NW
    n_ch = b_per_w // _CH
    mesh = plsc.VectorSubcoreMesh(core_axis_name="c", subcore_axis_name="s")

    @functools.partial(
        pl.kernel,
        mesh=mesh,
        out_type=jax.ShapeDtypeStruct((N, D), jnp.float32),
        scratch_types=[
            pltpu.VMEM((n_ch, _CH), jnp.int32),
            pltpu.VMEM((_NBUF, _CH, D), jnp.float32),
        ]
        + [pltpu.SemaphoreType.DMA] * (2 * _NBUF),
        compiler_params=pltpu.CompilerParams(use_tc_tiling_on_sc=False),
    )
    def gather_kernel(idx_hbm, table_hbm, out_hbm, idx_all, rows, *sems):
        sem_g, sem_w = sems[:_NBUF], sems[_NBUF:]
        wid = lax.axis_index("s") * NC + lax.axis_index("c")
        base = wid * b_per_w
        pltpu.sync_copy(idx_hbm.at[wid], idx_all)

        gathers = {}
        for b in range(min(_NBUF, n_ch)):
            gathers[b] = pltpu.async_copy(
                table_hbm.at[idx_all.at[b]], rows.at[b], sem_g[b])
        for i in range(n_ch):
            b = i % _NBUF
            gathers[i].wait()
            wb = pltpu.async_copy(
                rows.at[b], out_hbm.at[pl.ds(base + i * _CH, _CH)], sem_w[b])
            nxt = i + _NBUF
            wb.wait()
            if nxt < n_ch:
                gathers[nxt] = pltpu.async_copy(
                    table_hbm.at[idx_all.at[nxt]], rows.at[b], sem_g[b])

    return gather_kernel


def _retile_body(in_ref, out_ref):
    x = in_ref[...]
    y = jnp.transpose(x, (1, 0))
    D = x.shape[0]
    g = 128 // D
    y3 = y.reshape(y.shape[0] // g, g, D)
    for j in range(g):
        out_ref[:, j * D:(j + 1) * D] = y3[:, j, :]


@functools.cache
def _make_retile(V, D):
    v_blk = 16384
    n_blk = -(-V // v_blk)
    R = v_blk * D // 128
    return pl.pallas_call(
        _retile_body,
        grid=(n_blk,),
        in_specs=[pl.BlockSpec((D, v_blk), lambda i: (0, i))],
        out_specs=pl.BlockSpec((R, 128), lambda i: (i, 0)),
        out_shape=jax.ShapeDtypeStruct((n_blk * R, 128), jnp.float32),
    )


def _unflatten_body(in_ref, out_ref):
    z = in_ref[...]
    D = out_ref.shape[1]
    g = 128 // D
    S = out_ref.shape[2] // g
    for j in range(g):
        out_ref[0, :, j * S:(j + 1) * S] = jnp.transpose(
            z[:, j * D:(j + 1) * D], (1, 0))


@functools.cache
def _make_plane_transpose(B, F, D):
    rows = B * D // 128
    return pl.pallas_call(
        _unflatten_body,
        grid=(F,),
        in_specs=[pl.BlockSpec((rows, 128), lambda f: (f, 0))],
        out_specs=pl.BlockSpec((1, D, B), lambda f: (f, 0, 0)),
        out_shape=jax.ShapeDtypeStruct((F, D, B), jnp.float32),
    )


def kernel(token_ids, weight):
    B, F = token_ids.shape
    V, D = weight.shape
    N = B * F
    info = plsc.get_sparse_core_info()
    NW = info.num_cores * info.num_subcores
    g = 128 // D
    tid = token_ids.T.reshape(F, g, B // g).swapaxes(1, 2).reshape(F, B)
    idx = tid.reshape(NW, N // (NW * _CH), _CH)
    table = _make_retile(V, D)(weight.T)
    table = table.reshape(table.shape[0] * 128 // D, D)
    flat = _make_gather(table.shape[0], D, N)(idx, table)
    planes = _make_plane_transpose(B, F, D)(flat.reshape(N * D // 128, 128))
    return planes.transpose(2, 0, 1)
